# baseline (speedup 1.0000x reference)
import functools

import jax
import jax.numpy as jnp
import numpy as np
from jax import lax
from jax.experimental import pallas as pl
from jax.experimental.pallas import tpu as pltpu
from jax.experimental.pallas import tpu_sc as plsc

_FIELD_DIMS = [3846] * 25 + [3850]
_F = 26
_D = 16
_V = 100000
_OFFS = np.array((0, *np.cumsum(_FIELD_DIMS)[:-1]), dtype=np.int32)
_B = 4096
_PAIRS = [(i, j) for i in range(_F) for j in range(i + 1, _F)]
_NP = len(_PAIRS)

_VPAD = 100352
_VI = _VPAD // 2
_VC = _VI // 16
_SLABC = 136
_Q = 4
_NQ = _D // _Q
_NG = _B // 16

_NW = 32

_BASE = [i * (2 * _F - 1 - i) // 2 for i in range(_F)]

_mesh = plsc.VectorSubcoreMesh(core_axis_name="c", subcore_axis_name="s")


@functools.partial(
    pl.kernel,
    out_type=jax.ShapeDtypeStruct((_NP, 2, _B // 128, 8, 128), jnp.float32),
    mesh=_mesh,
    compiler_params=pltpu.CompilerParams(use_tc_tiling_on_sc=False,
                                         needs_layout_passes=False),
    scratch_types=[
        pltpu.VMEM((2 * _B,), jnp.int32),
        pltpu.VMEM((2 * _B,), jnp.int32),
        pltpu.VMEM((2, _Q, _SLABC, 16), jnp.int32),
        pltpu.VMEM((2, _Q, _SLABC, 16), jnp.int32),
        pltpu.VMEM((2, _B // 128, _Q, 128), jnp.float32),
        pltpu.SemaphoreType.DMA,
        pltpu.SemaphoreType.DMA,
        pltpu.SemaphoreType.DMA,
        pltpu.SemaphoreType.DMA,
        pltpu.SemaphoreType.DMA,
        pltpu.SemaphoreType.DMA,
    ],
)
def _ffm_sc(xadj_hbm, wt_hbm, ot_hbm, ia_v, ib_v,
            sa_v, sb_v, out_v, isem0, isem1, ssem0, ssem1, osem0, osem1):
    isem = (isem0, isem1)
    ssem = (ssem0, ssem1)
    osem = (osem0, osem1)
    wid = lax.axis_index("s") * 2 + lax.axis_index("c")
    p_lo = wid * _NP // _NW
    p_hi = (wid + 1) * _NP // _NW

    def params(k):
        i = jnp.int32(0)
        for t in range(1, _F):
            i = i + jnp.where(k >= _BASE[t], 1, 0).astype(jnp.int32)
        j = k - i * (2 * _F - 1 - i) // 2 + i + 1
        ea = jnp.bitwise_and(3846 * j, -256)
        eb = jnp.bitwise_and(3846 * i, -256)
        sa = pl.multiple_of(lax.shift_right_logical(ea, 5), 8)
        sb = pl.multiple_of(lax.shift_right_logical(eb, 5), 8)
        return i, j, sa, 3846 * j - ea, j, i, sb, 3846 * i - eb

    def fetch_idx(fa, fb, kk):
        sl = pl.ds(kk * _B, _B)
        pltpu.async_copy(xadj_hbm.at[fa], ia_v.at[sl], isem[kk])
        pltpu.async_copy(xadj_hbm.at[fb], ib_v.at[sl], isem[kk])

    def wait_idx(kk):
        sl = pl.ds(kk * _B, _B)
        pltpu.make_async_copy(xadj_hbm.at[0], ia_v.at[sl], isem[kk]).wait()
        pltpu.make_async_copy(xadj_hbm.at[0], ib_v.at[sl], isem[kk]).wait()

    def fire_slabs(pa, sa, pb, sb, q, slot):
        rows = pl.ds(q * _Q, _Q)
        pltpu.async_copy(wt_hbm.at[pa, rows, pl.ds(sa, _SLABC), :],
                         sa_v.at[slot], ssem[slot])
        pltpu.async_copy(wt_hbm.at[pb, rows, pl.ds(sb, _SLABC), :],
                         sb_v.at[slot], ssem[slot])

    def drain_slabs(slot):
        dummy = wt_hbm.at[0, pl.ds(0, _Q), pl.ds(0, _SLABC), :]
        pltpu.make_async_copy(dummy, sa_v.at[slot], ssem[slot]).wait()
        pltpu.make_async_copy(dummy, sb_v.at[slot], ssem[slot]).wait()

    def drain_out(slot):
        dummy = ot_hbm.at[0, 0, :, pl.ds(0, _Q), :]
        pltpu.make_async_copy(dummy, out_v.at[slot], osem[slot]).wait()

    def pair_body(k, kk):
        wait_idx(kk)

        nxt = jnp.minimum(k + 1, _NP - 1)
        npa, nfa, nsa, _, npb, nfb, nsb, _ = params(nxt)

        @pl.when(k + 1 < p_hi)
        def _():
            fetch_idx(nfa, nfb, 1 - kk)

        pa, fa, sa, aa, pb, fb, sb, ab = params(k)
        del fa, fb
        aav = jnp.full((16,), aa, jnp.int32)
        abv = jnp.full((16,), ab, jnp.int32)
        himask = jnp.full((16,), jnp.int32(-65536))

        for q in range(_NQ):
            drain_slabs(q % 2)
            if q < _NQ - 1:
                fire_slabs(pa, sa, pb, sb, q + 1, (q + 1) % 2)
            else:
                @pl.when(k + 1 < p_hi)
                def _():
                    fire_slabs(npa, nsa, npb, nsb, 0, 0)

            @pl.when((k - p_lo) * _NQ + q >= 2)
            def _():
                drain_out(q % 2)

            @plsc.parallel_loop(0, _NG, unroll=4)
            def _grp(g):
                iva = ia_v[pl.ds(kk * _B + g * 16, 16)] + aav
                ivb = ib_v[pl.ds(kk * _B + g * 16, 16)] + abv
                ca = lax.shift_right_logical(iva, 5)
                la = jnp.bitwise_and(lax.shift_right_logical(iva, 1), 15)
                oa = jnp.bitwise_and(iva, 1) == 1
                cb = lax.shift_right_logical(ivb, 5)
                lb = jnp.bitwise_and(lax.shift_right_logical(ivb, 1), 15)
                ob = jnp.bitwise_and(ivb, 1) == 1
                b1 = g // 8
                b0 = (g % 8) * 16
                for d in range(_Q):
                    pa_ = plsc.load_gather(sa_v.at[q % 2, d], [ca, la])
                    pb_ = plsc.load_gather(sb_v.at[q % 2, d], [cb, lb])
                    abits = jnp.where(oa, jnp.bitwise_and(pa_, himask),
                                      lax.shift_left(pa_, 16))
                    bbits = jnp.where(ob, jnp.bitwise_and(pb_, himask),
                                      lax.shift_left(pb_, 16))
                    af = plsc.bitcast(abits, jnp.float32)
                    bf = plsc.bitcast(bbits, jnp.float32)
                    out_v[q % 2, b1, d, pl.ds(b0, 16)] = af * bf

            pltpu.async_copy(out_v.at[q % 2],
                             ot_hbm.at[k, q // 2, :, pl.ds((q % 2) * _Q, _Q), :],
                             osem[q % 2])

    pa0, fa0, sa0, _, pb0, fb0, sb0, _ = params(p_lo)
    fetch_idx(fa0, fb0, 0)
    fire_slabs(pa0, sa0, pb0, sb0, 0, 0)

    @pl.loop(p_lo, p_hi, step=2)
    def _pairs(gg):
        pair_body(gg, 0)

        @pl.when(gg + 1 < p_hi)
        def _():
            pair_body(gg + 1, 1)

    drain_out(0)
    drain_out(1)


def kernel(x, W):
    wt = jnp.transpose(W, (0, 2, 1)).astype(jnp.bfloat16)
    wt = jnp.pad(wt, ((0, 0), (0, 0), (0, _VPAD - _V)))
    w32 = lax.bitcast_convert_type(wt.reshape(_F, _D, _VI, 2), jnp.int32)
    w4 = w32.reshape(_F, _D, _VC, 16)
    xadj = x.T
    ot5 = _ffm_sc(xadj, w4)
    return jnp.transpose(ot5, (2, 4, 0, 1, 3)).reshape(_B, _NP, _D)


# speedup vs baseline: 4.6031x; 4.6031x over previous
import functools

import jax
import jax.numpy as jnp
import numpy as np
from jax import lax
from jax.experimental import pallas as pl
from jax.experimental.pallas import tpu as pltpu
from jax.experimental.pallas import tpu_sc as plsc

_FIELD_DIMS = [3846] * 25 + [3850]
_F = 26
_D = 16
_V = 100000
_OFFS = np.array((0, *np.cumsum(_FIELD_DIMS)[:-1]), dtype=np.int32)
_B = 4096
_PAIRS = [(i, j) for i in range(_F) for j in range(i + 1, _F)]
_NP = len(_PAIRS)

_SLABW = 3856
_Q = 4
_NQ = _D // _Q
_NG = _B // 16

_NW = 32

_BASE = [i * (2 * _F - 1 - i) // 2 for i in range(_F)]
_ADJ = (_OFFS % 8).astype(np.int32)

_mesh = plsc.VectorSubcoreMesh(core_axis_name="c", subcore_axis_name="s")


@functools.partial(
    pl.kernel,
    out_type=jax.ShapeDtypeStruct((_NP, 2, _B // 128, 8, 128), jnp.float32),
    mesh=_mesh,
    compiler_params=pltpu.CompilerParams(use_tc_tiling_on_sc=False,
                                         needs_layout_passes=False),
    scratch_types=[
        pltpu.VMEM((2 * _B,), jnp.int32),
        pltpu.VMEM((2 * _B,), jnp.int32),
        pltpu.VMEM((2, _Q, _SLABW), jnp.float32),
        pltpu.VMEM((2, _Q, _SLABW), jnp.float32),
        pltpu.VMEM((2, _B // 128, _Q, 128), jnp.float32),
        pltpu.SemaphoreType.DMA,
        pltpu.SemaphoreType.DMA,
        pltpu.SemaphoreType.DMA,
        pltpu.SemaphoreType.DMA,
        pltpu.SemaphoreType.DMA,
        pltpu.SemaphoreType.DMA,
    ],
)
def _ffm_sc(xadj_hbm, wt_hbm, ot_hbm, ia_v, ib_v,
            sa_v, sb_v, out_v, isem0, isem1, ssem0, ssem1, osem0, osem1):
    isem = (isem0, isem1)
    ssem = (ssem0, ssem1)
    osem = (osem0, osem1)
    wid = lax.axis_index("s") * 2 + lax.axis_index("c")
    p_lo = wid * _NP // _NW
    p_hi = (wid + 1) * _NP // _NW

    def params(k):
        i = jnp.int32(0)
        for t in range(1, _F):
            i = i + jnp.where(k >= _BASE[t], 1, 0).astype(jnp.int32)
        j = k - i * (2 * _F - 1 - i) // 2 + i + 1
        sa = pl.multiple_of(jnp.bitwise_and(3846 * j, -8), 8)
        sb = pl.multiple_of(jnp.bitwise_and(3846 * i, -8), 8)
        return i, j, sa, j, i, sb

    def fetch_idx(fa, fb, kk):
        sl = pl.ds(kk * _B, _B)
        pltpu.async_copy(xadj_hbm.at[fa], ia_v.at[sl], isem[kk])
        pltpu.async_copy(xadj_hbm.at[fb], ib_v.at[sl], isem[kk])

    def wait_idx(kk):
        sl = pl.ds(kk * _B, _B)
        pltpu.make_async_copy(xadj_hbm.at[0], ia_v.at[sl], isem[kk]).wait()
        pltpu.make_async_copy(xadj_hbm.at[0], ib_v.at[sl], isem[kk]).wait()

    def fire_slabs(pa, sa, pb, sb, q, slot):
        rows = pl.ds(q * _Q, _Q)
        pltpu.async_copy(wt_hbm.at[pa, rows, pl.ds(sa, _SLABW)],
                         sa_v.at[slot], ssem[slot])
        pltpu.async_copy(wt_hbm.at[pb, rows, pl.ds(sb, _SLABW)],
                         sb_v.at[slot], ssem[slot])

    def drain_slabs(slot):
        dummy = wt_hbm.at[0, pl.ds(0, _Q), pl.ds(0, _SLABW)]
        pltpu.make_async_copy(dummy, sa_v.at[slot], ssem[slot]).wait()
        pltpu.make_async_copy(dummy, sb_v.at[slot], ssem[slot]).wait()

    def drain_out(slot):
        dummy = ot_hbm.at[0, 0, :, pl.ds(0, _Q), :]
        pltpu.make_async_copy(dummy, out_v.at[slot], osem[slot]).wait()

    def pair_body(k, kk):
        wait_idx(kk)

        nxt = jnp.minimum(k + 1, _NP - 1)
        npa, nfa, nsa, npb, nfb, nsb = params(nxt)

        @pl.when(k + 1 < p_hi)
        def _():
            fetch_idx(nfa, nfb, 1 - kk)

        pa, fa, sa, pb, fb, sb = params(k)
        del fa, fb

        for q in range(_NQ):
            drain_slabs(q % 2)
            if q < _NQ - 1:
                fire_slabs(pa, sa, pb, sb, q + 1, (q + 1) % 2)
            else:
                @pl.when(k + 1 < p_hi)
                def _():
                    fire_slabs(npa, nsa, npb, nsb, 0, 0)

            @pl.when((k - p_lo) * _NQ + q >= 2)
            def _():
                drain_out(q % 2)

            @plsc.parallel_loop(0, _NG, unroll=4)
            def _grp(g):
                iva = ia_v[pl.ds(kk * _B + g * 16, 16)]
                ivb = ib_v[pl.ds(kk * _B + g * 16, 16)]
                b1 = g // 8
                b0 = (g % 8) * 16
                for d in range(_Q):
                    a = plsc.load_gather(sa_v.at[q % 2, d], [iva])
                    b = plsc.load_gather(sb_v.at[q % 2, d], [ivb])
                    out_v[q % 2, b1, d, pl.ds(b0, 16)] = a * b

            pltpu.async_copy(out_v.at[q % 2],
                             ot_hbm.at[k, q // 2, :, pl.ds((q % 2) * _Q, _Q), :],
                             osem[q % 2])

    pa0, fa0, sa0, pb0, fb0, sb0 = params(p_lo)
    fetch_idx(fa0, fb0, 0)
    fire_slabs(pa0, sa0, pb0, sb0, 0, 0)

    @pl.loop(p_lo, p_hi, step=2)
    def _pairs(gg):
        pair_body(gg, 0)

        @pl.when(gg + 1 < p_hi)
        def _():
            pair_body(gg + 1, 1)

    drain_out(0)
    drain_out(1)


def kernel(x, W):
    wt = jnp.transpose(W, (0, 2, 1))
    xadj = x.T + jnp.asarray(_ADJ)[:, None]
    ot5 = _ffm_sc(xadj, wt)
    return jnp.transpose(ot5, (2, 4, 0, 1, 3)).reshape(_B, _NP, _D)


# speedup vs baseline: 4.6261x; 1.0050x over previous
import functools

import jax
import jax.numpy as jnp
import numpy as np
from jax import lax
from jax.experimental import pallas as pl
from jax.experimental.pallas import tpu as pltpu
from jax.experimental.pallas import tpu_sc as plsc

_FIELD_DIMS = [3846] * 25 + [3850]
_F = 26
_D = 16
_V = 100000
_OFFS = np.array((0, *np.cumsum(_FIELD_DIMS)[:-1]), dtype=np.int32)
_B = 4096
_PAIRS = [(i, j) for i in range(_F) for j in range(i + 1, _F)]
_NP = len(_PAIRS)

_SLABW = 3856
_Q = 4
_NQ = _D // _Q
_NG = _B // 16

_NW = 32

_BASE = [i * (2 * _F - 1 - i) // 2 for i in range(_F)]
_ADJ = (_OFFS % 8).astype(np.int32)

_mesh = plsc.VectorSubcoreMesh(core_axis_name="c", subcore_axis_name="s")


@functools.partial(
    pl.kernel,
    out_type=jax.ShapeDtypeStruct((_NP, 2, _B // 128, 8, 128), jnp.float32),
    mesh=_mesh,
    compiler_params=pltpu.CompilerParams(use_tc_tiling_on_sc=False,
                                         needs_layout_passes=False),
    scratch_types=[
        pltpu.VMEM((2 * _B,), jnp.int32),
        pltpu.VMEM((2 * _B,), jnp.int32),
        pltpu.VMEM((2, _Q, _SLABW), jnp.float32),
        pltpu.VMEM((2, _Q, _SLABW), jnp.float32),
        pltpu.VMEM((2, _B // 128, _Q, 128), jnp.float32),
        pltpu.SemaphoreType.DMA,
        pltpu.SemaphoreType.DMA,
        pltpu.SemaphoreType.DMA,
        pltpu.SemaphoreType.DMA,
        pltpu.SemaphoreType.DMA,
        pltpu.SemaphoreType.DMA,
    ],
)
def _ffm_sc(xadj_hbm, wt_hbm, ot_hbm, ia_v, ib_v,
            sa_v, sb_v, out_v, isem0, isem1, ssem0, ssem1, osem0, osem1):
    isem = (isem0, isem1)
    ssem = (ssem0, ssem1)
    osem = (osem0, osem1)
    wid = lax.axis_index("s") * 2 + lax.axis_index("c")
    p_lo = wid * _NP // _NW
    p_hi = (wid + 1) * _NP // _NW

    def params(k):
        i = jnp.int32(0)
        for t in range(1, _F):
            i = i + jnp.where(k >= _BASE[t], 1, 0).astype(jnp.int32)
        j = k - i * (2 * _F - 1 - i) // 2 + i + 1
        sa = pl.multiple_of(jnp.bitwise_and(3846 * j, -8), 8)
        sb = pl.multiple_of(jnp.bitwise_and(3846 * i, -8), 8)
        return i, j, sa, j, i, sb

    def fetch_idx(fa, fb, kk):
        sl = pl.ds(kk * _B, _B)
        pltpu.async_copy(xadj_hbm.at[fa], ia_v.at[sl], isem[kk])
        pltpu.async_copy(xadj_hbm.at[fb], ib_v.at[sl], isem[kk])

    def wait_idx(kk):
        sl = pl.ds(kk * _B, _B)
        pltpu.make_async_copy(xadj_hbm.at[0], ia_v.at[sl], isem[kk]).wait()
        pltpu.make_async_copy(xadj_hbm.at[0], ib_v.at[sl], isem[kk]).wait()

    def fire_slabs(pa, sa, pb, sb, q, slot):
        rows = pl.ds(q * _Q, _Q)
        pltpu.async_copy(wt_hbm.at[pa, rows, pl.ds(sa, _SLABW)],
                         sa_v.at[slot], ssem[slot])
        pltpu.async_copy(wt_hbm.at[pb, rows, pl.ds(sb, _SLABW)],
                         sb_v.at[slot], ssem[slot])

    def drain_slabs(slot):
        dummy = wt_hbm.at[0, pl.ds(0, _Q), pl.ds(0, _SLABW)]
        pltpu.make_async_copy(dummy, sa_v.at[slot], ssem[slot]).wait()
        pltpu.make_async_copy(dummy, sb_v.at[slot], ssem[slot]).wait()

    def drain_out(slot):
        dummy = ot_hbm.at[0, 0, :, pl.ds(0, _Q), :]
        pltpu.make_async_copy(dummy, out_v.at[slot], osem[slot]).wait()

    def pair_body(k, kk):
        wait_idx(kk)

        nxt = jnp.minimum(k + 1, _NP - 1)
        npa, nfa, nsa, npb, nfb, nsb = params(nxt)

        @pl.when(k + 1 < p_hi)
        def _():
            fetch_idx(nfa, nfb, 1 - kk)

        pa, fa, sa, pb, fb, sb = params(k)
        del fa, fb

        for q in range(_NQ):
            if q < _NQ - 1:
                fire_slabs(pa, sa, pb, sb, q + 1, (q + 1) % 2)
            else:
                @pl.when(k + 1 < p_hi)
                def _():
                    fire_slabs(npa, nsa, npb, nsb, 0, 0)
            drain_slabs(q % 2)

            @pl.when((k - p_lo) * _NQ + q >= 2)
            def _():
                drain_out(q % 2)

            @plsc.parallel_loop(0, _NG, unroll=4)
            def _grp(g):
                iva = ia_v[pl.ds(kk * _B + g * 16, 16)]
                ivb = ib_v[pl.ds(kk * _B + g * 16, 16)]
                b1 = g // 8
                b0 = (g % 8) * 16
                for d in range(_Q):
                    a = plsc.load_gather(sa_v.at[q % 2, d], [iva])
                    b = plsc.load_gather(sb_v.at[q % 2, d], [ivb])
                    out_v[q % 2, b1, d, pl.ds(b0, 16)] = a * b

            pltpu.async_copy(out_v.at[q % 2],
                             ot_hbm.at[k, q // 2, :, pl.ds((q % 2) * _Q, _Q), :],
                             osem[q % 2])

    pa0, fa0, sa0, pb0, fb0, sb0 = params(p_lo)
    fetch_idx(fa0, fb0, 0)
    fire_slabs(pa0, sa0, pb0, sb0, 0, 0)

    @pl.loop(p_lo, p_hi, step=2)
    def _pairs(gg):
        pair_body(gg, 0)

        @pl.when(gg + 1 < p_hi)
        def _():
            pair_body(gg + 1, 1)

    drain_out(0)
    drain_out(1)


def kernel(x, W):
    wt = jnp.transpose(W, (0, 2, 1))
    xadj = x.T + jnp.asarray(_ADJ)[:, None]
    ot5 = _ffm_sc(xadj, wt)
    return jnp.transpose(ot5, (2, 4, 0, 1, 3)).reshape(_B, _NP, _D)
